# Initial kernel scaffold; baseline (speedup 1.0000x reference)
import jax
import jax.numpy as jnp
from jax.experimental import pallas as pl


def kernel(node_feats, edge_index, history_feats, W_self, W_neigh, b_sage, W_ih, W_hh, b_ih, b_hh, W_cls, b_cls):
    raise NotImplementedError("write your pallas kernel here")


# speedup vs baseline: 5.3525x; 5.3525x over previous
import functools

import jax
import jax.numpy as jnp
from jax import lax
from jax.experimental import pallas as pl
from jax.experimental.pallas import tpu as pltpu
from jax.experimental.pallas import tpu_sc as plsc

N = 10000
E = 320000
D = 128
H = 32
OUT = 32
T = 20

_NC = 2
_NS = 16
_NW = _NC * _NS
_C = 80
_EPW = E // _NW
_RPS = 1000
_NZ = N // _RPS


def _proj_body(nf_ref, wself_ref, wneigh_ref, wcls_ref, ps_ref, pn_ref):
    wcls = wcls_ref[...]
    wsc = lax.dot_general(wcls, wself_ref[...], (((1,), (0,)), ((), ())),
                          preferred_element_type=jnp.float32)
    wnc = lax.dot_general(wcls, wneigh_ref[...], (((1,), (0,)), ((), ())),
                          preferred_element_type=jnp.float32)
    x = nf_ref[...]
    ps_ref[...] = lax.dot_general(x, wsc, (((1,), (1,)), ((), ())),
                                  preferred_element_type=jnp.float32)
    pn_ref[...] = lax.dot_general(x, wnc, (((1,), (1,)), ((), ())),
                                  preferred_element_type=jnp.float32)


def _lstm_body(hist_ref, wih_ref, whh_ref, b_ref, wcls_ref, out_ref):
    bn = hist_ref.shape[0]
    wih = wih_ref[...]
    whh = whh_ref[...]
    b = b_ref[...]
    h = jnp.zeros((bn, H), jnp.float32)
    c = jnp.zeros((bn, H), jnp.float32)
    for t in range(T):
        x_t = hist_ref[:, t, :]
        gates = (lax.dot_general(x_t, wih, (((1,), (1,)), ((), ())),
                                 preferred_element_type=jnp.float32)
                 + lax.dot_general(h, whh, (((1,), (1,)), ((), ())),
                                   preferred_element_type=jnp.float32)
                 + b)
        i = jax.nn.sigmoid(gates[:, :H])
        f = jax.nn.sigmoid(gates[:, H:2 * H])
        g = jnp.tanh(gates[:, 2 * H:3 * H])
        o = jax.nn.sigmoid(gates[:, 3 * H:])
        c = f * c + i * g
        h = o * jnp.tanh(c)
    out_ref[...] = lax.dot_general(h, wcls_ref[...], (((1,), (1,)), ((), ())),
                                   preferred_element_type=jnp.float32)


def _combine_body(ps_ref, ho_ref, acc_ref, deg_ref, bsage_ref, wcls_ref,
                  bcls_ref, out_ref):
    acc = acc_ref[0] + acc_ref[1]
    deg = deg_ref[0, :, 0:1] + deg_ref[1, :, 0:1]
    neigh = acc / jnp.maximum(deg, 1.0)
    bsc = lax.dot_general(bsage_ref[...], wcls_ref[...], (((1,), (1,)), ((), ())),
                          preferred_element_type=jnp.float32)
    out_ref[...] = ps_ref[...] + neigh + ho_ref[...] + bsc + bcls_ref[...]


def _sc_agg_body(pn_hbm, src_hbm, dst_hbm, z32_hbm, z16_hbm, ones_hbm,
                 acc_out, deg_out,
                 src_v, dst_v, rows_v, ones_v, acc_sh, deg_sh, sem):
    c = lax.axis_index("c")
    s = lax.axis_index("s")
    wid = s * _NC + c
    @pl.when(s < _NZ)
    def _zero():
        pltpu.sync_copy(z32_hbm.at[pl.ds(s * _RPS, _RPS)],
                        acc_sh.at[pl.ds(s * _RPS, _RPS)])
        pltpu.sync_copy(z16_hbm.at[pl.ds(s * _RPS, _RPS)],
                        deg_sh.at[pl.ds(s * _RPS, _RPS)])

    pltpu.sync_copy(ones_hbm, ones_v)
    plsc.subcore_barrier()

    base = wid * _EPW

    @pl.loop(0, _EPW // _C)
    def _chunk(g):
        off = base + g * _C
        pltpu.sync_copy(src_hbm.at[pl.ds(off, _C)], src_v)
        pltpu.sync_copy(dst_hbm.at[pl.ds(off, _C)], dst_v)
        pltpu.async_copy(pn_hbm.at[src_v], rows_v, sem).wait()
        pltpu.sync_copy(rows_v, acc_sh.at[dst_v], add=True)
        pltpu.sync_copy(ones_v, deg_sh.at[dst_v], add=True)

    plsc.subcore_barrier()

    @pl.when(s < _NZ)
    def _writeback():
        pltpu.sync_copy(acc_sh.at[pl.ds(s * _RPS, _RPS)],
                        acc_out.at[c, pl.ds(s * _RPS, _RPS)])
        pltpu.sync_copy(deg_sh.at[pl.ds(s * _RPS, _RPS)],
                        deg_out.at[c, pl.ds(s * _RPS, _RPS)])


def _sc_aggregate(pn, src, dst):
    z32 = jnp.zeros((N, H), jnp.float32)
    z16 = jnp.zeros((N, 16), jnp.float32)
    ones = jnp.zeros((_C, 16), jnp.float32).at[:, 0].set(1.0)
    mesh = plsc.VectorSubcoreMesh(core_axis_name="c", subcore_axis_name="s")
    f = pl.kernel(
        _sc_agg_body,
        out_type=[
            jax.ShapeDtypeStruct((_NC, N, H), jnp.float32),
            jax.ShapeDtypeStruct((_NC, N, 16), jnp.float32),
        ],
        mesh=mesh,
        scratch_types=[
            pltpu.VMEM((_C,), jnp.int32),
            pltpu.VMEM((_C,), jnp.int32),
            pltpu.VMEM((_C, H), jnp.float32),
            pltpu.VMEM((_C, 16), jnp.float32),
            pltpu.VMEM_SHARED((N, H), jnp.float32),
            pltpu.VMEM_SHARED((N, 16), jnp.float32),
            pltpu.SemaphoreType.DMA,
        ],
        compiler_params=pltpu.CompilerParams(use_tc_tiling_on_sc=False),
    )
    return f(pn, src, dst, z32, z16, ones)


_BN = 400


def _tc_proj(node_feats, W_self, W_neigh, W_cls):
    grid = (N // _BN,)
    return pl.pallas_call(
        _proj_body,
        grid=grid,
        in_specs=[
            pl.BlockSpec((_BN, D), lambda i: (i, 0)),
            pl.BlockSpec((H, D), lambda i: (0, 0)),
            pl.BlockSpec((H, D), lambda i: (0, 0)),
            pl.BlockSpec((OUT, H), lambda i: (0, 0)),
        ],
        out_specs=[
            pl.BlockSpec((_BN, OUT), lambda i: (i, 0)),
            pl.BlockSpec((_BN, OUT), lambda i: (i, 0)),
        ],
        out_shape=[
            jax.ShapeDtypeStruct((N, OUT), jnp.float32),
            jax.ShapeDtypeStruct((N, OUT), jnp.float32),
        ],
    )(node_feats, W_self, W_neigh, W_cls)


def _tc_lstm(history_feats, W_ih, W_hh, b, W_cls):
    grid = (N // _BN,)
    return pl.pallas_call(
        _lstm_body,
        grid=grid,
        in_specs=[
            pl.BlockSpec((_BN, T, H), lambda i: (i, 0, 0)),
            pl.BlockSpec((4 * H, H), lambda i: (0, 0)),
            pl.BlockSpec((4 * H, H), lambda i: (0, 0)),
            pl.BlockSpec((1, 4 * H), lambda i: (0, 0)),
            pl.BlockSpec((OUT, H), lambda i: (0, 0)),
        ],
        out_specs=pl.BlockSpec((_BN, OUT), lambda i: (i, 0)),
        out_shape=jax.ShapeDtypeStruct((N, OUT), jnp.float32),
    )(history_feats, W_ih, W_hh, b, W_cls)


def _tc_combine(ps, ho, acc, deg, b_sage, W_cls, b_cls):
    grid = (N // _BN,)
    return pl.pallas_call(
        _combine_body,
        grid=grid,
        in_specs=[
            pl.BlockSpec((_BN, OUT), lambda i: (i, 0)),
            pl.BlockSpec((_BN, OUT), lambda i: (i, 0)),
            pl.BlockSpec((_NC, _BN, H), lambda i: (0, i, 0)),
            pl.BlockSpec((_NC, _BN, 16), lambda i: (0, i, 0)),
            pl.BlockSpec((1, H), lambda i: (0, 0)),
            pl.BlockSpec((OUT, H), lambda i: (0, 0)),
            pl.BlockSpec((1, OUT), lambda i: (0, 0)),
        ],
        out_specs=pl.BlockSpec((_BN, OUT), lambda i: (i, 0)),
        out_shape=jax.ShapeDtypeStruct((N, OUT), jnp.float32),
    )(ps, ho, acc, deg, b_sage, W_cls, b_cls)


def kernel(node_feats, edge_index, history_feats, W_self, W_neigh, b_sage,
           W_ih, W_hh, b_ih, b_hh, W_cls, b_cls):
    src = edge_index[0].astype(jnp.int32)
    dst = edge_index[1].astype(jnp.int32)
    b = (b_ih + b_hh).reshape(1, 4 * H)

    ps, pn = _tc_proj(node_feats, W_self, W_neigh, W_cls)
    acc, deg = _sc_aggregate(pn, src, dst)
    ho = _tc_lstm(history_feats, W_ih, W_hh, b, W_cls)
    return _tc_combine(ps, ho, acc, deg, b_sage.reshape(1, H), W_cls,
                       b_cls.reshape(1, OUT))
